# baseline (speedup 1.0000x reference)
import jax
import jax.numpy as jnp
from jax.experimental import pallas as pl
from jax.experimental.pallas import tpu as pltpu

H = 1024; E = 8; F = 512; FS = 1024; N = 2048
RSF = 2.5
ET = E + 2
LW = 128
FBLK = 512
NFT = N // FBLK

_NEG = -1e30


def _route_block(x, gw, bias):
    logits = jax.lax.dot_general(
        x, gw, (((1,), (0,)), ((), ())), preferred_element_type=jnp.float32)
    lane = jax.lax.broadcasted_iota(jnp.int32, (FBLK, LW), 1)
    valid = lane < E
    scores = jax.nn.sigmoid(logits)
    sfc = scores + bias
    sfc_m = jnp.where(valid, sfc, 0.0)
    r1 = pltpu.roll(sfc_m, LW - 1, 1)
    r2 = pltpu.roll(sfc_m, 1, 1)
    gs = sfc_m + jnp.where(lane % 2 == 0, r1, r2)
    gid = lane // 2
    grp = jnp.where(valid & (lane % 2 == 0), gs, _NEG)
    m1 = jnp.max(grp, axis=1, keepdims=True)
    g1 = jnp.min(jnp.where(grp == m1, gid, 999), axis=1, keepdims=True)
    grp2 = jnp.where(gid == g1, _NEG, grp)
    m2 = jnp.max(grp2, axis=1, keepdims=True)
    g2 = jnp.min(jnp.where(grp2 == m2, gid, 999), axis=1, keepdims=True)
    chosen = (gid == g1) | (gid == g2)
    tmp = jnp.where(chosen & valid, sfc, 0.0)
    tmp = jnp.where(valid, tmp, _NEG)
    M1 = jnp.max(tmp, axis=1, keepdims=True)
    e1 = jnp.min(jnp.where(tmp == M1, lane, 999), axis=1, keepdims=True)
    tmp2 = jnp.where(lane == e1, _NEG, tmp)
    M2 = jnp.max(tmp2, axis=1, keepdims=True)
    e2 = jnp.min(jnp.where(tmp2 == M2, lane, 999), axis=1, keepdims=True)
    w1 = jnp.sum(jnp.where(lane == e1, sfc, 0.0), axis=1, keepdims=True)
    w2 = jnp.sum(jnp.where(lane == e2, sfc, 0.0), axis=1, keepdims=True)
    den = w1 + w2 + 1e-20
    return (jnp.where(lane == e1, w1 / den * RSF, 0.0)
            + jnp.where(lane == e2, w2 / den * RSF, 0.0)
            + jnp.where((lane >= E) & (lane < ET), 1.0, 0.0))


def _mlp(x, wg, wu, wd):
    g = jnp.dot(x, wg, preferred_element_type=jnp.float32)
    u = jnp.dot(x, wu, preferred_element_type=jnp.float32)
    h = (g * jax.nn.sigmoid(g) * u).astype(jnp.bfloat16)
    return jnp.dot(h, wd, preferred_element_type=jnp.float32)


def _body(x_ref, gw_ref, b_ref,
          wgA_ref, wuA_ref, wdA_ref,
          wgB_ref, wuB_ref, wdB_ref,
          sg_ref, su_ref, sd_ref,
          o_ref, w8_ref):
    e = pl.program_id(0)
    t = pl.program_id(1)
    rows = pl.ds(t * FBLK, FBLK)
    x = x_ref[rows, :]

    @pl.when(e == 0)
    def _route():
        w8_ref[rows, :] = _route_block(x, gw_ref[...], b_ref[...])

    lane = jax.lax.broadcasted_iota(jnp.int32, (FBLK, LW), 1)
    scale = jnp.sum(jnp.where(lane == e, w8_ref[rows, :], 0.0),
                    axis=1, keepdims=True)

    @pl.when(e == 0)
    def _init():
        o_ref[rows, :] = _mlp(x, wgA_ref[0], wuA_ref[0], wdA_ref[0]) * scale

    @pl.when((e != 0) & (e < E) & (e % 2 == 0))
    def _even():
        o_ref[rows, :] += _mlp(x, wgA_ref[0], wuA_ref[0], wdA_ref[0]) * scale

    @pl.when((e < E) & (e % 2 == 1))
    def _odd():
        o_ref[rows, :] += _mlp(x, wgB_ref[0], wuB_ref[0], wdB_ref[0]) * scale

    @pl.when(e >= E)
    def _shared():
        o_ref[rows, :] += _mlp(x, sg_ref[...], su_ref[...], sd_ref[...])


def kernel(x, gate_w, correction_bias, w_gate, w_up, w_down,
           s_gate, s_up, s_down):
    flat = x.reshape(N, H)
    gwp = jnp.zeros((H, LW), jnp.float32).at[:, :E].set(gate_w)
    bp = jnp.zeros((1, LW), jnp.float32).at[0, :E].set(correction_bias)
    eA = lambda e: jnp.minimum(2 * ((e + 1) // 2), E - 2)
    eB = lambda e: jnp.minimum(2 * (e // 2) + 1, E - 1)
    sh = lambda e: jnp.clip(e - E, 0, 1)
    y = pl.pallas_call(
        _body,
        grid=(ET, NFT),
        in_specs=[
            pl.BlockSpec((N, H), lambda e, t: (0, 0)),
            pl.BlockSpec((H, LW), lambda e, t: (0, 0)),
            pl.BlockSpec((1, LW), lambda e, t: (0, 0)),
            pl.BlockSpec((1, H, F), lambda e, t: (eA(e), 0, 0)),
            pl.BlockSpec((1, H, F), lambda e, t: (eA(e), 0, 0)),
            pl.BlockSpec((1, F, H), lambda e, t: (eA(e), 0, 0)),
            pl.BlockSpec((1, H, F), lambda e, t: (eB(e), 0, 0)),
            pl.BlockSpec((1, H, F), lambda e, t: (eB(e), 0, 0)),
            pl.BlockSpec((1, F, H), lambda e, t: (eB(e), 0, 0)),
            pl.BlockSpec((H, F), lambda e, t: (0, sh(e))),
            pl.BlockSpec((H, F), lambda e, t: (0, sh(e))),
            pl.BlockSpec((F, H), lambda e, t: (sh(e), 0)),
        ],
        out_specs=pl.BlockSpec((N, H), lambda e, t: (0, 0)),
        out_shape=jax.ShapeDtypeStruct((N, H), jnp.float32),
        scratch_shapes=[pltpu.VMEM((N, LW), jnp.float32)],
        compiler_params=pltpu.CompilerParams(
            dimension_semantics=("arbitrary", "arbitrary")),
    )(flat, gwp, bp,
      w_gate, w_up, w_down,
      w_gate, w_up, w_down,
      s_gate, s_up, s_down)
    return y.reshape(1, N, H)


# speedup vs baseline: 1.0113x; 1.0113x over previous
import jax
import jax.numpy as jnp
from jax.experimental import pallas as pl
from jax.experimental.pallas import tpu as pltpu

H = 1024; E = 8; F = 512; FS = 1024; N = 2048
RSF = 2.5
ET = E + 2
LW = 128
FBLK = 512
NFT = N // FBLK

_NEG = -1e30


def _route_block(x, gw, bias):
    logits = jax.lax.dot_general(
        x, gw, (((1,), (0,)), ((), ())), preferred_element_type=jnp.float32)
    lane = jax.lax.broadcasted_iota(jnp.int32, (FBLK, LW), 1)
    valid = lane < E
    scores = jax.nn.sigmoid(logits)
    sfc = scores + bias
    sfc_m = jnp.where(valid, sfc, 0.0)
    r1 = pltpu.roll(sfc_m, LW - 1, 1)
    r2 = pltpu.roll(sfc_m, 1, 1)
    gs = sfc_m + jnp.where(lane % 2 == 0, r1, r2)
    gid = lane // 2
    grp = jnp.where(valid & (lane % 2 == 0), gs, _NEG)
    m1 = jnp.max(grp, axis=1, keepdims=True)
    g1 = jnp.min(jnp.where(grp == m1, gid, 999), axis=1, keepdims=True)
    grp2 = jnp.where(gid == g1, _NEG, grp)
    m2 = jnp.max(grp2, axis=1, keepdims=True)
    g2 = jnp.min(jnp.where(grp2 == m2, gid, 999), axis=1, keepdims=True)
    chosen = (gid == g1) | (gid == g2)
    tmp = jnp.where(chosen & valid, sfc, 0.0)
    tmp = jnp.where(valid, tmp, _NEG)
    M1 = jnp.max(tmp, axis=1, keepdims=True)
    e1 = jnp.min(jnp.where(tmp == M1, lane, 999), axis=1, keepdims=True)
    tmp2 = jnp.where(lane == e1, _NEG, tmp)
    M2 = jnp.max(tmp2, axis=1, keepdims=True)
    e2 = jnp.min(jnp.where(tmp2 == M2, lane, 999), axis=1, keepdims=True)
    w1 = jnp.sum(jnp.where(lane == e1, sfc, 0.0), axis=1, keepdims=True)
    w2 = jnp.sum(jnp.where(lane == e2, sfc, 0.0), axis=1, keepdims=True)
    den = w1 + w2 + 1e-20
    return (jnp.where(lane == e1, w1 / den * RSF, 0.0)
            + jnp.where(lane == e2, w2 / den * RSF, 0.0)
            + jnp.where((lane >= E) & (lane < ET), 1.0, 0.0))


def _mlp(x, wg, wu, wd):
    g = jnp.dot(x, wg, preferred_element_type=jnp.float32)
    u = jnp.dot(x, wu, preferred_element_type=jnp.float32)
    h = g * jax.nn.sigmoid(g) * u
    return jnp.dot(h, wd, preferred_element_type=jnp.float32)


def _body(x_ref, gw_ref, b_ref,
          wgA_ref, wuA_ref, wdA_ref,
          wgB_ref, wuB_ref, wdB_ref,
          sg_ref, su_ref, sd_ref,
          o_ref, w8_ref):
    e = pl.program_id(0)
    t = pl.program_id(1)
    rows = pl.ds(t * FBLK, FBLK)
    x = x_ref[rows, :]

    @pl.when(e == 0)
    def _route():
        w8_ref[rows, :] = _route_block(x, gw_ref[...], b_ref[...])

    lane = jax.lax.broadcasted_iota(jnp.int32, (FBLK, LW), 1)
    scale = jnp.sum(jnp.where(lane == e, w8_ref[rows, :], 0.0),
                    axis=1, keepdims=True)

    @pl.when(e == 0)
    def _init():
        o_ref[rows, :] = _mlp(x, wgA_ref[0], wuA_ref[0], wdA_ref[0]) * scale

    @pl.when((e != 0) & (e < E) & (e % 2 == 0))
    def _even():
        o_ref[rows, :] += _mlp(x, wgA_ref[0], wuA_ref[0], wdA_ref[0]) * scale

    @pl.when((e < E) & (e % 2 == 1))
    def _odd():
        o_ref[rows, :] += _mlp(x, wgB_ref[0], wuB_ref[0], wdB_ref[0]) * scale

    @pl.when(e >= E)
    def _shared():
        o_ref[rows, :] += _mlp(x, sg_ref[...], su_ref[...], sd_ref[...])


def kernel(x, gate_w, correction_bias, w_gate, w_up, w_down,
           s_gate, s_up, s_down):
    flat = x.reshape(N, H)
    gwp = jnp.zeros((H, LW), jnp.float32).at[:, :E].set(gate_w)
    bp = jnp.zeros((1, LW), jnp.float32).at[0, :E].set(correction_bias)
    eA = lambda e: jnp.minimum(2 * ((e + 1) // 2), E - 2)
    eB = lambda e: jnp.minimum(2 * (e // 2) + 1, E - 1)
    sh = lambda e: jnp.clip(e - E, 0, 1)
    y = pl.pallas_call(
        _body,
        grid=(ET, NFT),
        in_specs=[
            pl.BlockSpec((N, H), lambda e, t: (0, 0)),
            pl.BlockSpec((H, LW), lambda e, t: (0, 0)),
            pl.BlockSpec((1, LW), lambda e, t: (0, 0)),
            pl.BlockSpec((1, H, F), lambda e, t: (eA(e), 0, 0)),
            pl.BlockSpec((1, H, F), lambda e, t: (eA(e), 0, 0)),
            pl.BlockSpec((1, F, H), lambda e, t: (eA(e), 0, 0)),
            pl.BlockSpec((1, H, F), lambda e, t: (eB(e), 0, 0)),
            pl.BlockSpec((1, H, F), lambda e, t: (eB(e), 0, 0)),
            pl.BlockSpec((1, F, H), lambda e, t: (eB(e), 0, 0)),
            pl.BlockSpec((H, F), lambda e, t: (0, sh(e))),
            pl.BlockSpec((H, F), lambda e, t: (0, sh(e))),
            pl.BlockSpec((F, H), lambda e, t: (sh(e), 0)),
        ],
        out_specs=pl.BlockSpec((N, H), lambda e, t: (0, 0)),
        out_shape=jax.ShapeDtypeStruct((N, H), jnp.float32),
        scratch_shapes=[pltpu.VMEM((N, LW), jnp.float32)],
        compiler_params=pltpu.CompilerParams(
            dimension_semantics=("arbitrary", "arbitrary")),
    )(flat, gwp, bp,
      w_gate, w_up, w_down,
      w_gate, w_up, w_down,
      s_gate, s_up, s_down)
    return y.reshape(1, N, H)
